# Initial kernel scaffold; baseline (speedup 1.0000x reference)
import jax
import jax.numpy as jnp
from jax.experimental import pallas as pl


def kernel(h_pair, pair_vu_idx, pair_uw_idx, pair_vw_idx, geom_features, psi_W1, psi_b1, psi_W2, psi_b2, phi_W1, phi_b1, phi_W2, phi_b2):
    raise NotImplementedError("write your pallas kernel here")


# speedup vs baseline: 3.7215x; 3.7215x over previous
import functools

import jax
import jax.numpy as jnp
from jax import lax
from jax.experimental import pallas as pl
from jax.experimental.pallas import tpu as pltpu
from jax.experimental.pallas import tpu_sc as plsc

P = 160000
T = 320000
D = 128
NC = 2
NS = 16
NW = NC * NS

f32 = jnp.float32

_BM = 4000


def _mm1_body(h_ref, wa_ref, wb_ref, wc_ref, a_ref, b_ref, c_ref):
    h = h_ref[...]
    a_ref[...] = jnp.dot(h, wa_ref[...], preferred_element_type=f32)
    b_ref[...] = jnp.dot(h, wb_ref[...], preferred_element_type=f32)
    c_ref[...] = jnp.dot(h, wc_ref[...], preferred_element_type=f32)


def _premul(h_pair, w1a, w1b, w1c):
    grid = (P // _BM,)
    wspec = pl.BlockSpec((D, D), lambda i: (0, 0))
    rspec = pl.BlockSpec((_BM, D), lambda i: (i, 0))
    return pl.pallas_call(
        _mm1_body,
        grid=grid,
        in_specs=[rspec, wspec, wspec, wspec],
        out_specs=[rspec, rspec, rspec],
        out_shape=[jax.ShapeDtypeStruct((P, D), f32)] * 3,
    )(h_pair, w1a, w1b, w1c)


_GB = 200
_RW = T // NW
_NGB = _RW // _GB


def _gather_body(a, b, c, ivu, iuw, ivw, ga, gb, gc, ibuf, buf, sem):
    cid = lax.axis_index("c")
    sid = lax.axis_index("s")
    wid = sid * NC + cid
    base0 = wid * _RW

    def step(i, carry):
        base = base0 + i * _GB
        for idx_hbm, tbl, out in ((ivu, a, ga), (iuw, b, gb), (ivw, c, gc)):
            pltpu.sync_copy(idx_hbm.at[pl.ds(base, _GB)], ibuf)
            pltpu.async_copy(tbl.at[ibuf], buf, sem).wait()
            pltpu.sync_copy(buf, out.at[pl.ds(base, _GB)])
        return carry

    lax.fori_loop(0, _NGB, step, 0)


_gather3 = pl.kernel(
    _gather_body,
    out_type=[jax.ShapeDtypeStruct((T, D), f32)] * 3,
    mesh=plsc.VectorSubcoreMesh(core_axis_name="c", subcore_axis_name="s"),
    scratch_types=[
        pltpu.VMEM((_GB,), jnp.int32),
        pltpu.VMEM((_GB, D), f32),
        pltpu.SemaphoreType.DMA,
    ],
)


_BT = 4000


def _psi_body(ga_ref, gb_ref, gc_ref, g_ref, w1d_ref, b1_ref, w2_ref, b2_ref,
              m_ref):
    pre = (ga_ref[...] + gb_ref[...] + gc_ref[...]
           + jnp.dot(g_ref[...], w1d_ref[...], preferred_element_type=f32)
           + b1_ref[...])
    h = pre * jax.nn.sigmoid(pre)
    m_ref[...] = jnp.dot(h, w2_ref[...], preferred_element_type=f32) + b2_ref[...]


def _psi(ga, gb, gc, geom, w1d, b1, w2, b2):
    grid = (T // _BT,)
    rspec = pl.BlockSpec((_BT, D), lambda i: (i, 0))
    return pl.pallas_call(
        _psi_body,
        grid=grid,
        in_specs=[rspec, rspec, rspec,
                  pl.BlockSpec((_BT, 4), lambda i: (i, 0)),
                  pl.BlockSpec((4, D), lambda i: (0, 0)),
                  pl.BlockSpec((1, D), lambda i: (0, 0)),
                  pl.BlockSpec((D, D), lambda i: (0, 0)),
                  pl.BlockSpec((1, D), lambda i: (0, 0))],
        out_specs=rspec,
        out_shape=jax.ShapeDtypeStruct((T, D), f32),
    )(ga, gb, gc, geom, w1d, b1, w2, b2)


_CH = 8192
_NCHUNK = 20
_PPAD = _NCHUNK * _CH
_NPASS = _NCHUNK // NC
_SS = T // NS
_NV = _SS // 16
_BB = 128
_RPT = _CH // NS
_LCAP = _SS + 2 * _BB


def _scatter_body(m, ivw, zeros, agg,
                  idxbuf, plist, gbuf, dbuf, mbuf, table, sem):
    cid = lax.axis_index("c")
    sid = lax.axis_index("s")
    pltpu.sync_copy(ivw.at[pl.ds(sid * _SS, _SS)], idxbuf)
    zero16 = jnp.full((16,), 0, jnp.int32)
    one16 = jnp.full((16,), 1, jnp.int32)
    pad16 = jnp.full((16,), _CH, jnp.int32)
    m14 = jnp.full((16,), (1 << 14) - 1, jnp.int32)
    c14 = jnp.full((16,), 14, jnp.int32)
    sbase = jnp.full((16,), sid * _SS, jnp.int32)
    lane = lax.iota(jnp.int32, 16)

    def do_pass(p, carry):
        chunk = p * NC + cid
        lo = chunk * _CH
        pltpu.sync_copy(zeros, table.at[pl.ds(sid * _RPT, _RPT)])
        plsc.subcore_barrier()

        lo_v = jnp.full((16,), lo, jnp.int32)
        hi_v = lo_v + _CH

        def scan(i, cnt):
            v = idxbuf[pl.ds(i * 16, 16)]
            msk = (v >= lo_v) & (v < hi_v)
            key = jnp.where(msk, zero16, one16)
            pv = ((lane + jnp.full((16,), i * 16, jnp.int32)) << c14) | \
                 ((v - lo_v) & m14)
            _, sv = plsc.sort_key_val(key, pv)
            plist[pl.ds(cnt, 16)] = sv
            return cnt + plsc.all_reduce_population_count(msk)[0]

        cnt = lax.fori_loop(0, _NV, scan, jnp.int32(0))

        for j in range(_BB // 16):
            plist[pl.ds(cnt + j * 16, 16)] = pad16
        nb = (cnt + _BB - 1) // _BB

        def batch(bi, carry2):
            for j in range(_BB // 16):
                pv = plist[pl.ds(bi * _BB + j * 16, 16)]
                gbuf[pl.ds(j * 16, 16)] = \
                    lax.shift_right_logical(pv, c14) + sbase
                dbuf[pl.ds(j * 16, 16)] = pv & m14
            pltpu.async_copy(m.at[gbuf], mbuf, sem).wait()
            pltpu.sync_copy(mbuf, table.at[dbuf], add=True)
            return carry2

        lax.fori_loop(0, nb, batch, 0)
        plsc.subcore_barrier()
        pltpu.sync_copy(table.at[pl.ds(sid * _RPT, _RPT)],
                        agg.at[pl.ds(lo + sid * _RPT, _RPT)])
        plsc.subcore_barrier()
        return carry

    lax.fori_loop(0, _NPASS, do_pass, 0)


_scatter = pl.kernel(
    _scatter_body,
    out_type=jax.ShapeDtypeStruct((_PPAD, D), f32),
    mesh=plsc.VectorSubcoreMesh(core_axis_name="c", subcore_axis_name="s"),
    compiler_params=pltpu.CompilerParams(needs_layout_passes=False),
    scratch_types=[
        pltpu.VMEM((_SS,), jnp.int32),
        pltpu.VMEM((_LCAP,), jnp.int32),
        pltpu.VMEM((_BB,), jnp.int32),
        pltpu.VMEM((_BB,), jnp.int32),
        pltpu.VMEM((_BB, D), f32),
        pltpu.VMEM_SHARED((_CH + 8, D), f32),
        pltpu.SemaphoreType.DMA,
    ],
)


def _phi_body(h_ref, agg_ref, wa_ref, wb_ref, b1_ref, w2_ref, b2_ref, o_ref):
    h = h_ref[...]
    pre = (jnp.dot(h, wa_ref[...], preferred_element_type=f32)
           + jnp.dot(agg_ref[...], wb_ref[...], preferred_element_type=f32)
           + b1_ref[...])
    z = pre * jax.nn.sigmoid(pre)
    o_ref[...] = h + jnp.dot(z, w2_ref[...], preferred_element_type=f32) + b2_ref[...]


def _phi(h_pair, agg, wa, wb, b1, w2, b2):
    grid = (P // _BM,)
    rspec = pl.BlockSpec((_BM, D), lambda i: (i, 0))
    wspec = pl.BlockSpec((D, D), lambda i: (0, 0))
    bspec = pl.BlockSpec((1, D), lambda i: (0, 0))
    return pl.pallas_call(
        _phi_body,
        grid=grid,
        in_specs=[rspec, rspec, wspec, wspec, bspec, wspec, bspec],
        out_specs=rspec,
        out_shape=jax.ShapeDtypeStruct((P, D), f32),
    )(h_pair, agg, wa, wb, b1, w2, b2)


def kernel(h_pair, pair_vu_idx, pair_uw_idx, pair_vw_idx, geom_features,
           psi_W1, psi_b1, psi_W2, psi_b2, phi_W1, phi_b1, phi_W2, phi_b2):
    ivu = pair_vu_idx.astype(jnp.int32)
    iuw = pair_uw_idx.astype(jnp.int32)
    ivw = pair_vw_idx.astype(jnp.int32)

    w1a = psi_W1[0:D]
    w1b = psi_W1[D:2 * D]
    w1c = psi_W1[2 * D:3 * D]
    w1d = psi_W1[3 * D:]

    a, b, c = _premul(h_pair, w1a, w1b, w1c)
    ga, gb, gc = _gather3(a, b, c, ivu, iuw, ivw)
    m = _psi(ga, gb, gc, geom_features,
             w1d, psi_b1.reshape(1, D), psi_W2, psi_b2.reshape(1, D))
    zeros = jnp.zeros((_RPT, D), f32)
    agg = _scatter(m, ivw, zeros)
    out = _phi(h_pair, agg,
               phi_W1[0:D], phi_W1[D:2 * D],
               phi_b1.reshape(1, D), phi_W2, phi_b2.reshape(1, D))
    return out


# speedup vs baseline: 3.8656x; 1.0387x over previous
import functools

import jax
import jax.numpy as jnp
from jax import lax
from jax.experimental import pallas as pl
from jax.experimental.pallas import tpu as pltpu
from jax.experimental.pallas import tpu_sc as plsc

P = 160000
T = 320000
D = 128
NC = 2
NS = 16
NW = NC * NS

f32 = jnp.float32

_BM = 4000


def _mm1_body(h_ref, w_ref, o_ref):
    o_ref[...] = jnp.dot(h_ref[...], w_ref[0], preferred_element_type=f32)


def _premul(h_pair, w_stack):
    nb = P // _BM
    return pl.pallas_call(
        _mm1_body,
        grid=(3, nb),
        in_specs=[pl.BlockSpec((_BM, D), lambda s, i: (i, 0)),
                  pl.BlockSpec((1, D, D), lambda s, i: (s, 0, 0))],
        out_specs=pl.BlockSpec((_BM, D), lambda s, i: (s * nb + i, 0)),
        out_shape=jax.ShapeDtypeStruct((3 * P, D), f32),
    )(h_pair, w_stack)


_GB = 400
_RW = 3 * T // NW
_NGB = _RW // _GB


def _gather_body(tab, iall, out, ibuf0, ibuf1, buf0, buf1, sem0, sem1, osem):
    cid = lax.axis_index("c")
    sid = lax.axis_index("s")
    wid = sid * NC + cid
    base0 = wid * _RW
    ibufs = (ibuf0, ibuf1)
    bufs = (buf0, buf1)
    sems = (sem0, sem1)

    pltpu.sync_copy(iall.at[pl.ds(base0, _GB)], ibuf0)
    cp0 = pltpu.async_copy(tab.at[ibuf0], buf0, sem0)

    def step(i, carry):
        for b in range(2):
            g = i + b
            pltpu.make_async_copy(tab.at[ibufs[b]], bufs[b], sems[b]).wait()

            @pl.when(g + 1 < _NGB)
            def _start_next():
                nxt = 1 - b
                pltpu.sync_copy(iall.at[pl.ds(base0 + (g + 1) * _GB, _GB)],
                                ibufs[nxt])
                pltpu.async_copy(tab.at[ibufs[nxt]], bufs[nxt], sems[nxt])

            pltpu.sync_copy(bufs[b], out.at[pl.ds(base0 + g * _GB, _GB)])
        return carry

    lax.fori_loop(0, _NGB // 2, lambda i, c: step(i * 2, c), 0)
    if _NGB % 2:
        g = _NGB - 1
        pltpu.make_async_copy(tab.at[ibuf0], buf0, sem0).wait()
        pltpu.sync_copy(buf0, out.at[pl.ds(base0 + g * _GB, _GB)])


_gather3 = pl.kernel(
    _gather_body,
    out_type=jax.ShapeDtypeStruct((3 * T, D), f32),
    mesh=plsc.VectorSubcoreMesh(core_axis_name="c", subcore_axis_name="s"),
    compiler_params=pltpu.CompilerParams(needs_layout_passes=False),
    scratch_types=[
        pltpu.VMEM((_GB,), jnp.int32),
        pltpu.VMEM((_GB,), jnp.int32),
        pltpu.VMEM((_GB, D), f32),
        pltpu.VMEM((_GB, D), f32),
        pltpu.SemaphoreType.DMA,
        pltpu.SemaphoreType.DMA,
        pltpu.SemaphoreType.DMA,
    ],
)


_BT = 4000


def _psi_body(ga_ref, gb_ref, gc_ref, g_ref, w1d_ref, b1_ref, w2_ref, b2_ref,
              m_ref):
    pre = (ga_ref[...] + gb_ref[...] + gc_ref[...]
           + jnp.dot(g_ref[...], w1d_ref[...], preferred_element_type=f32)
           + b1_ref[...])
    h = pre * jax.nn.sigmoid(pre)
    m_ref[...] = jnp.dot(h, w2_ref[...], preferred_element_type=f32) + b2_ref[...]


def _psi(gall, geom, w1d, b1, w2, b2):
    grid = (T // _BT,)
    nb = T // _BT
    rspec = pl.BlockSpec((_BT, D), lambda i: (i, 0))
    return pl.pallas_call(
        _psi_body,
        grid=grid,
        in_specs=[rspec,
                  pl.BlockSpec((_BT, D), lambda i: (nb + i, 0)),
                  pl.BlockSpec((_BT, D), lambda i: (2 * nb + i, 0)),
                  pl.BlockSpec((_BT, 4), lambda i: (i, 0)),
                  pl.BlockSpec((4, D), lambda i: (0, 0)),
                  pl.BlockSpec((1, D), lambda i: (0, 0)),
                  pl.BlockSpec((D, D), lambda i: (0, 0)),
                  pl.BlockSpec((1, D), lambda i: (0, 0))],
        out_specs=rspec,
        out_shape=jax.ShapeDtypeStruct((T, D), f32),
    )(gall, gall, gall, geom, w1d, b1, w2, b2)


_CH = 8192
_NCHUNK = 20
_PPAD = _NCHUNK * _CH
_NPASS = _NCHUNK // NC
_SS = T // NS
_NV = _SS // 16
_BB = 128
_RPT = _CH // NS
_LCAP = _SS + 2 * _BB


def _scatter_body(m, ivw, zeros, agg,
                  idxbuf, plist, gbuf, dbuf, mbuf, table, sem):
    cid = lax.axis_index("c")
    sid = lax.axis_index("s")
    pltpu.sync_copy(ivw.at[pl.ds(sid * _SS, _SS)], idxbuf)
    zero16 = jnp.full((16,), 0, jnp.int32)
    one16 = jnp.full((16,), 1, jnp.int32)
    pad16 = jnp.full((16,), _CH, jnp.int32)
    m14 = jnp.full((16,), (1 << 14) - 1, jnp.int32)
    c14 = jnp.full((16,), 14, jnp.int32)
    sbase = jnp.full((16,), sid * _SS, jnp.int32)
    lane = lax.iota(jnp.int32, 16)

    def do_pass(p, carry):
        chunk = p * NC + cid
        lo = chunk * _CH
        pltpu.sync_copy(zeros, table.at[pl.ds(sid * _RPT, _RPT)])
        plsc.subcore_barrier()

        lo_v = jnp.full((16,), lo, jnp.int32)
        hi_v = lo_v + _CH

        def scan(i, cnt):
            v = idxbuf[pl.ds(i * 16, 16)]
            msk = (v >= lo_v) & (v < hi_v)
            key = jnp.where(msk, zero16, one16)
            pv = ((lane + jnp.full((16,), i * 16, jnp.int32)) << c14) | \
                 ((v - lo_v) & m14)
            _, sv = plsc.sort_key_val(key, pv)
            plist[pl.ds(cnt, 16)] = sv
            return cnt + plsc.all_reduce_population_count(msk)[0]

        cnt = lax.fori_loop(0, _NV, scan, jnp.int32(0))

        for j in range(_BB // 16):
            plist[pl.ds(cnt + j * 16, 16)] = pad16
        nb = (cnt + _BB - 1) // _BB

        def batch(bi, carry2):
            for j in range(_BB // 16):
                pv = plist[pl.ds(bi * _BB + j * 16, 16)]
                gbuf[pl.ds(j * 16, 16)] = \
                    lax.shift_right_logical(pv, c14) + sbase
                dbuf[pl.ds(j * 16, 16)] = pv & m14
            pltpu.async_copy(m.at[gbuf], mbuf, sem).wait()
            pltpu.sync_copy(mbuf, table.at[dbuf], add=True)
            return carry2

        lax.fori_loop(0, nb, batch, 0)
        plsc.subcore_barrier()
        pltpu.sync_copy(table.at[pl.ds(sid * _RPT, _RPT)],
                        agg.at[pl.ds(lo + sid * _RPT, _RPT)])
        plsc.subcore_barrier()
        return carry

    lax.fori_loop(0, _NPASS, do_pass, 0)


_scatter = pl.kernel(
    _scatter_body,
    out_type=jax.ShapeDtypeStruct((_PPAD, D), f32),
    mesh=plsc.VectorSubcoreMesh(core_axis_name="c", subcore_axis_name="s"),
    compiler_params=pltpu.CompilerParams(needs_layout_passes=False),
    scratch_types=[
        pltpu.VMEM((_SS,), jnp.int32),
        pltpu.VMEM((_LCAP,), jnp.int32),
        pltpu.VMEM((_BB,), jnp.int32),
        pltpu.VMEM((_BB,), jnp.int32),
        pltpu.VMEM((_BB, D), f32),
        pltpu.VMEM_SHARED((_CH + 8, D), f32),
        pltpu.SemaphoreType.DMA,
    ],
)


def _phi_body(h_ref, agg_ref, wa_ref, wb_ref, b1_ref, w2_ref, b2_ref, o_ref):
    h = h_ref[...]
    pre = (jnp.dot(h, wa_ref[...], preferred_element_type=f32)
           + jnp.dot(agg_ref[...], wb_ref[...], preferred_element_type=f32)
           + b1_ref[...])
    z = pre * jax.nn.sigmoid(pre)
    o_ref[...] = h + jnp.dot(z, w2_ref[...], preferred_element_type=f32) + b2_ref[...]


def _phi(h_pair, agg, wa, wb, b1, w2, b2):
    grid = (P // _BM,)
    rspec = pl.BlockSpec((_BM, D), lambda i: (i, 0))
    wspec = pl.BlockSpec((D, D), lambda i: (0, 0))
    bspec = pl.BlockSpec((1, D), lambda i: (0, 0))
    return pl.pallas_call(
        _phi_body,
        grid=grid,
        in_specs=[rspec, rspec, wspec, wspec, bspec, wspec, bspec],
        out_specs=rspec,
        out_shape=jax.ShapeDtypeStruct((P, D), f32),
    )(h_pair, agg, wa, wb, b1, w2, b2)


def kernel(h_pair, pair_vu_idx, pair_uw_idx, pair_vw_idx, geom_features,
           psi_W1, psi_b1, psi_W2, psi_b2, phi_W1, phi_b1, phi_W2, phi_b2):
    ivu = pair_vu_idx.astype(jnp.int32)
    iuw = pair_uw_idx.astype(jnp.int32)
    ivw = pair_vw_idx.astype(jnp.int32)

    w_stack = psi_W1[:3 * D].reshape(3, D, D)
    w1d = psi_W1[3 * D:]

    tab = _premul(h_pair, w_stack)
    iall = jnp.concatenate([ivu, iuw + P, ivw + 2 * P])
    gall = _gather3(tab, iall)
    m = _psi(gall, geom_features,
             w1d, psi_b1.reshape(1, D), psi_W2, psi_b2.reshape(1, D))
    zeros = jnp.zeros((_RPT, D), f32)
    agg = _scatter(m, ivw, zeros)
    out = _phi(h_pair, agg,
               phi_W1[0:D], phi_W1[D:2 * D],
               phi_b1.reshape(1, D), phi_W2, phi_b2.reshape(1, D))
    return out
